# baseline (speedup 1.0000x reference)
import jax
import jax.numpy as jnp
from jax.experimental import pallas as pl


def _conv2d(x, w, b, stride=1, padding=0):
    out = jax.lax.conv_general_dilated(
        x, w, window_strides=(stride, stride),
        padding=[(padding, padding), (padding, padding)],
        dimension_numbers=('NCHW', 'OIHW', 'NCHW'))
    return out + b[None, :, None, None]


def _round_bf16(x):
    u = jax.lax.bitcast_convert_type(x, jnp.int32)
    u = u + 0x7FFF + ((u >> 16) & 1)
    u = jax.lax.bitwise_and(u, jnp.int32(-65536))
    return jax.lax.bitcast_convert_type(u, jnp.float32)


def _pool2(x, hw, op):
    c = x.shape[-1]
    h = hw // 2
    x = x.reshape(h, 2, h, 2, c)
    return op(op(x, 3), 1).reshape(h * h, c)


def _attn_batch_kernel(theta_ref, phi_ref, gconv_ref, wmat_ref, wb_ref,
                       wy_ref, sum_ref, sq_ref):
    hi = jax.lax.Precision.HIGHEST
    g = _pool2(gconv_ref[0], 128, jnp.max)
    g = _round_bf16(g)
    g4 = g.reshape(32, 2, 32, 2, 32)
    gsum = jnp.sum(jnp.sum(g4, axis=3), axis=1).reshape(1024, 32)
    gtl = g4[:, 0, :, 0, :].reshape(1024, 32)
    phi = _pool2(phi_ref[0], 64, jnp.max)
    f = jax.lax.dot_general(theta_ref[0], phi,
                            (((1,), (1,)), ((), ())),
                            preferred_element_type=jnp.float32)
    m1 = jnp.max(f, axis=1, keepdims=True)
    is1 = f >= m1
    f2 = jnp.where(is1, -jnp.inf, f)
    m2 = jnp.max(f2, axis=1, keepdims=True)
    e2 = jnp.exp(m2 - m1)
    denom = 4.0 + e2
    w1 = _round_bf16(1.0 / denom)
    w2 = _round_bf16(e2 / denom)
    p1 = jnp.where(is1, w1, 0.0)
    p2 = jnp.where(f2 >= m2, w2, 0.0)
    y = (jnp.dot(p1, gsum, preferred_element_type=jnp.float32, precision=hi)
         + jnp.dot(p2, gtl, preferred_element_type=jnp.float32,
                   precision=hi))
    wy = jnp.dot(y.astype(jnp.bfloat16), wmat_ref[...].astype(jnp.bfloat16),
                 preferred_element_type=jnp.float32) + wb_ref[...]
    wy_ref[0] = wy
    s = jnp.sum(wy, axis=0, keepdims=True)
    q = jnp.sum(wy * wy, axis=0, keepdims=True)
    sum_ref[0] = jnp.broadcast_to(jnp.pad(s, ((0, 0), (0, 64))), (8, 128))
    sq_ref[0] = jnp.broadcast_to(jnp.pad(q, ((0, 0), (0, 64))), (8, 128))


def _bn_kernel(wy_ref, sum_ref, sq_ref, bnw_ref, bnb_ref, z_ref):
    nb = wy_ref.shape[0]
    count = float(nb * 1024)
    total = jnp.zeros((1, 64), dtype=jnp.float32)
    total_sq = jnp.zeros((1, 64), dtype=jnp.float32)
    for b in range(nb):
        total = total + sum_ref[b, 0:1, 0:64]
        total_sq = total_sq + sq_ref[b, 0:1, 0:64]
    mean = total / count
    var = total_sq / count - mean * mean
    inv = bnw_ref[...] * jax.lax.rsqrt(var + 1e-5)
    shift = bnb_ref[...] - mean * inv
    for b in range(nb):
        z_ref[b] = wy_ref[b] * inv + shift


def kernel(x_ms, hp_pan, x_pan, g_w, g_b, theta_w, theta_b,
           phi_w, phi_b, W_w, W_b, bn_w, bn_b):
    B = x_ms.shape[0]

    g_conv = _conv2d(hp_pan, g_w, g_b)
    theta_s = _conv2d(x_ms, theta_w, theta_b,
                      stride=2, padding=1)
    phi_conv = _conv2d(x_pan, phi_w, phi_b,
                       stride=2, padding=1)

    wy, psum, psq = pl.pallas_call(
        _attn_batch_kernel,
        grid=(B,),
        in_specs=[
            pl.BlockSpec((1, 1024, 32), lambda b: (b, 0, 0)),
            pl.BlockSpec((1, 4096, 32), lambda b: (b, 0, 0)),
            pl.BlockSpec((1, 16384, 32), lambda b: (b, 0, 0)),
            pl.BlockSpec((32, 64), lambda b: (0, 0)),
            pl.BlockSpec((1, 64), lambda b: (0, 0)),
        ],
        out_specs=[
            pl.BlockSpec((1, 1024, 64), lambda b: (b, 0, 0)),
            pl.BlockSpec((1, 8, 128), lambda b: (b, 0, 0)),
            pl.BlockSpec((1, 8, 128), lambda b: (b, 0, 0)),
        ],
        out_shape=[
            jax.ShapeDtypeStruct((B, 1024, 64), jnp.float32),
            jax.ShapeDtypeStruct((B, 8, 128), jnp.float32),
            jax.ShapeDtypeStruct((B, 8, 128), jnp.float32),
        ],
    )(theta_s.reshape(B, 32, 1024).transpose(0, 2, 1),
      phi_conv.reshape(B, 32, 4096).transpose(0, 2, 1),
      g_conv.reshape(B, 32, 16384).transpose(0, 2, 1),
      W_w.reshape(64, 32).T, W_b.reshape(1, 64))

    z_small = pl.pallas_call(
        _bn_kernel,
        out_shape=jax.ShapeDtypeStruct((B, 1024, 64), jnp.float32),
    )(wy, psum, psq, bn_w.reshape(1, 64), bn_b.reshape(1, 64))

    z = z_small.transpose(0, 2, 1).reshape(B, 64, 32, 32)
    z = jnp.repeat(jnp.repeat(z, 2, axis=2), 2, axis=3)
    return (z, x_pan)
